# baseline (speedup 1.0000x reference)
import functools

import jax
import jax.numpy as jnp
from jax import lax
from jax.experimental import pallas as pl
from jax.experimental.pallas import tpu as pltpu
from jax.experimental.pallas import tpu_sc as plsc

H = 4096
W = 8192
B = 1048576
NC = 2
NS = 16
NW = NC * NS
N_PER_W = B // NW
CHUNK = 4096
NCHUNK = N_PER_W // CHUNK
LANES = 16

MAGIC = 12582912.0


def _scale_round(x, scale, hi):
    t = x * jnp.float32(scale) + jnp.float32(scale + MAGIC)
    i = t.astype(jnp.int32) - jnp.int32(MAGIC)
    return jnp.minimum(jnp.maximum(i, 0), hi)


def _make_kernel():
    mesh = plsc.VectorSubcoreMesh(core_axis_name="c", subcore_axis_name="s")

    @functools.partial(
        pl.kernel,
        mesh=mesh,
        out_type=jax.ShapeDtypeStruct((B,), jnp.float32),
        scratch_types=[
            pltpu.VMEM((2 * N_PER_W,), jnp.float32),
            pltpu.VMEM((CHUNK,), jnp.int32),
            pltpu.VMEM((CHUNK,), jnp.int32),
            pltpu.VMEM((CHUNK,), jnp.int32),
            pltpu.VMEM((CHUNK,), jnp.int32),
            pltpu.VMEM((N_PER_W,), jnp.float32),
            pltpu.SemaphoreType.DMA,
            pltpu.SemaphoreType.DMA,
            pltpu.SemaphoreType.DMA,
            pltpu.SemaphoreType.DMA,
            pltpu.SemaphoreType.DMA,
            pltpu.SemaphoreType.DMA,
        ],
        compiler_params=pltpu.CompilerParams(
            needs_layout_passes=False, disable_bounds_checks=True),
    )
    def night_light(x_hbm, f_hbm, y_hbm,
                    xv, ix0, ix1, ix2, ix3, yall,
                    sx, sg0, sg1, sg2, sg3, sw):
        wid = lax.axis_index("s") * NC + lax.axis_index("c")
        base0 = wid * N_PER_W
        ix = [ix0, ix1, ix2, ix3]
        sg = [sg0, sg1, sg2, sg3]

        hx = pltpu.async_copy(
            x_hbm.at[pl.ds(2 * base0, 2 * N_PER_W)], xv, sx)

        def compute(k):
            b = k % 4

            @plsc.parallel_loop(0, CHUNK // 128, 1, unroll=2)
            def blk_body(blk):
                boff = (k * (CHUNK // 128) + blk) * 256
                for g in range(8):
                    x0 = xv[pl.ds(boff + g * LANES, LANES)]
                    x1 = xv[pl.ds(boff + 128 + g * LANES, LANES)]
                    col = _scale_round(x0, W // 2, W - 1)
                    row = _scale_round(x1, H // 2, H - 1)
                    p = ((row << 7) + (row >> 3) * 64512
                         + col + (col >> 7) * 896)
                    ix[b][pl.ds(blk * 128 + g * LANES, LANES)] = p

        def gather(k):
            return pltpu.async_copy(
                f_hbm.at[ix[k % 4]],
                yall.at[pl.ds(k * CHUNK, CHUNK)],
                sg[k % 4])

        hx.wait()
        hg = {}
        for k in range(NCHUNK):
            compute(k)
            if k >= 3:
                hg[k - 3].wait()
            hg[k] = gather(k)
        for k in range(NCHUNK - 3, NCHUNK):
            hg[k].wait()
        pltpu.async_copy(yall, y_hbm.at[pl.ds(base0, N_PER_W)], sw).wait()

    return night_light


_night_light = _make_kernel()


@jax.jit
def kernel(x, f):
    x_phys = x.reshape(B // 128, 128, 2).transpose(0, 2, 1).reshape(2 * B)
    f_phys = (
        f.reshape(H // 8, 8, W // 128, 128)
        .transpose(0, 2, 1, 3)
        .reshape(H * W)
    )
    return _night_light(x_phys, f_phys)


# speedup vs baseline: 1.0242x; 1.0242x over previous
import functools

import jax
import jax.numpy as jnp
from jax import lax
from jax.experimental import pallas as pl
from jax.experimental.pallas import tpu as pltpu
from jax.experimental.pallas import tpu_sc as plsc

H = 4096
W = 8192
B = 1048576
NC = 2
NS = 16
NW = NC * NS
N_PER_W = B // NW
CHUNK = 2048
NCHUNK = N_PER_W // CHUNK
LANES = 16

MAGIC = 12582912.0


def _scale_round(x, scale, hi):
    t = x * jnp.float32(scale) + jnp.float32(scale + MAGIC)
    i = t.astype(jnp.int32) - jnp.int32(MAGIC)
    return jnp.minimum(jnp.maximum(i, 0), hi)


def _make_kernel():
    mesh = plsc.VectorSubcoreMesh(core_axis_name="c", subcore_axis_name="s")

    @functools.partial(
        pl.kernel,
        mesh=mesh,
        out_type=jax.ShapeDtypeStruct((B,), jnp.float32),
        scratch_types=[
            pltpu.VMEM((2 * N_PER_W,), jnp.float32),
            pltpu.VMEM((CHUNK,), jnp.int32),
            pltpu.VMEM((CHUNK,), jnp.int32),
            pltpu.VMEM((CHUNK,), jnp.int32),
            pltpu.VMEM((CHUNK,), jnp.int32),
            pltpu.VMEM((N_PER_W,), jnp.float32),
            pltpu.SemaphoreType.DMA,
            pltpu.SemaphoreType.DMA,
            pltpu.SemaphoreType.DMA,
            pltpu.SemaphoreType.DMA,
            pltpu.SemaphoreType.DMA,
            pltpu.SemaphoreType.DMA,
        ],
        compiler_params=pltpu.CompilerParams(
            needs_layout_passes=False, disable_bounds_checks=True),
    )
    def night_light(x_hbm, f_hbm, y_hbm,
                    xv, ix0, ix1, ix2, ix3, yall,
                    sx, sg0, sg1, sg2, sg3, sw):
        wid = lax.axis_index("s") * NC + lax.axis_index("c")
        base0 = wid * N_PER_W
        ix = [ix0, ix1, ix2, ix3]
        sg = [sg0, sg1, sg2, sg3]

        hx = pltpu.async_copy(
            x_hbm.at[pl.ds(2 * base0, 2 * N_PER_W)], xv, sx)

        def compute(k):
            b = k % 4

            def pt_body(i, c2):
                j = k * CHUNK // LANES + i
                off = (j >> 3) * 256 + (j & 7) * LANES
                x0 = xv[pl.ds(off, LANES)]
                x1 = xv[pl.ds(off + 128, LANES)]
                col = _scale_round(x0, W // 2, W - 1)
                row = _scale_round(x1, H // 2, H - 1)
                p = ((row << 7) + (row >> 3) * 64512
                     + col + (col >> 7) * 896)
                ix[b][pl.ds(i * LANES, LANES)] = p
                return c2

            lax.fori_loop(0, CHUNK // LANES, pt_body, 0, unroll=4)

        def gather(k):
            return pltpu.async_copy(
                f_hbm.at[ix[k % 4]],
                yall.at[pl.ds(k * CHUNK, CHUNK)],
                sg[k % 4])

        def wb(k):
            return pltpu.async_copy(
                yall.at[pl.ds(k * CHUNK, CHUNK)],
                y_hbm.at[pl.ds(base0 + k * CHUNK, CHUNK)],
                sw)

        hx.wait()
        hg, hw = {}, {}
        for k in range(NCHUNK):
            compute(k)
            if k >= 3:
                hg[k - 3].wait()
                hw[k - 3] = wb(k - 3)
            hg[k] = gather(k)
        for k in range(NCHUNK - 3, NCHUNK):
            hg[k].wait()
            hw[k] = wb(k)
        for k in range(NCHUNK):
            hw[k].wait()

    return night_light


_night_light = _make_kernel()


@jax.jit
def kernel(x, f):
    x_phys = x.reshape(B // 128, 128, 2).transpose(0, 2, 1).reshape(2 * B)
    f_phys = (
        f.reshape(H // 8, 8, W // 128, 128)
        .transpose(0, 2, 1, 3)
        .reshape(H * W)
    )
    return _night_light(x_phys, f_phys)


# speedup vs baseline: 1.0243x; 1.0001x over previous
import functools

import jax
import jax.numpy as jnp
from jax import lax
from jax.experimental import pallas as pl
from jax.experimental.pallas import tpu as pltpu
from jax.experimental.pallas import tpu_sc as plsc

H = 4096
W = 8192
B = 1048576
NC = 2
NS = 16
NW = NC * NS
N_PER_W = B // NW
CHUNK = 8192
NCHUNK = N_PER_W // CHUNK
LANES = 16

MAGIC = 12582912.0


def _scale_round(x, scale, hi):
    t = x * jnp.float32(scale) + jnp.float32(scale + MAGIC)
    i = t.astype(jnp.int32) - jnp.int32(MAGIC)
    return jnp.minimum(jnp.maximum(i, 0), hi)


def _make_kernel():
    mesh = plsc.VectorSubcoreMesh(core_axis_name="c", subcore_axis_name="s")

    @functools.partial(
        pl.kernel,
        mesh=mesh,
        out_type=jax.ShapeDtypeStruct((B,), jnp.float32),
        scratch_types=[
            pltpu.VMEM((2 * N_PER_W,), jnp.float32),
            pltpu.VMEM((CHUNK,), jnp.int32),
            pltpu.VMEM((CHUNK,), jnp.int32),
            pltpu.VMEM((CHUNK,), jnp.int32),
            pltpu.VMEM((CHUNK,), jnp.int32),
            pltpu.VMEM((N_PER_W,), jnp.float32),
            pltpu.SemaphoreType.DMA,
            pltpu.SemaphoreType.DMA,
            pltpu.SemaphoreType.DMA,
            pltpu.SemaphoreType.DMA,
            pltpu.SemaphoreType.DMA,
            pltpu.SemaphoreType.DMA,
        ],
        compiler_params=pltpu.CompilerParams(
            needs_layout_passes=False, disable_bounds_checks=True),
    )
    def night_light(x_hbm, f_hbm, y_hbm,
                    xv, ix0, ix1, ix2, ix3, yall,
                    sx, sg0, sg1, sg2, sg3, sw):
        wid = lax.axis_index("s") * NC + lax.axis_index("c")
        base0 = wid * N_PER_W
        ix = [ix0, ix1, ix2, ix3]
        sg = [sg0, sg1, sg2, sg3]

        hx = pltpu.async_copy(
            x_hbm.at[pl.ds(2 * base0, 2 * N_PER_W)], xv, sx)

        def compute(k):
            b = k % 4

            def pt_body(i, c2):
                j = k * CHUNK // LANES + i
                off = (j >> 3) * 256 + (j & 7) * LANES
                x0 = xv[pl.ds(off, LANES)]
                x1 = xv[pl.ds(off + 128, LANES)]
                col = _scale_round(x0, W // 2, W - 1)
                row = _scale_round(x1, H // 2, H - 1)
                p = ((row << 7) + (row >> 3) * 64512
                     + col + (col >> 7) * 896)
                ix[b][pl.ds(i * LANES, LANES)] = p
                return c2

            lax.fori_loop(0, CHUNK // LANES, pt_body, 0, unroll=4)

        def gather(k):
            return pltpu.async_copy(
                f_hbm.at[ix[k % 4]],
                yall.at[pl.ds(k * CHUNK, CHUNK)],
                sg[k % 4])

        hx.wait()
        hg = {}
        for k in range(NCHUNK):
            compute(k)
            if k >= 3:
                hg[k - 3].wait()
            hg[k] = gather(k)
        for k in range(NCHUNK - 3, NCHUNK):
            hg[k].wait()
        pltpu.async_copy(yall, y_hbm.at[pl.ds(base0, N_PER_W)], sw).wait()

    return night_light


_night_light = _make_kernel()


@jax.jit
def kernel(x, f):
    x_phys = x.reshape(B // 128, 128, 2).transpose(0, 2, 1).reshape(2 * B)
    f_phys = (
        f.reshape(H // 8, 8, W // 128, 128)
        .transpose(0, 2, 1, 3)
        .reshape(H * W)
    )
    return _night_light(x_phys, f_phys)


# speedup vs baseline: 1.0490x; 1.0241x over previous
import functools

import jax
import jax.numpy as jnp
from jax import lax
from jax.experimental import pallas as pl
from jax.experimental.pallas import tpu as pltpu
from jax.experimental.pallas import tpu_sc as plsc

H = 4096
W = 8192
B = 1048576
NC = 2
NS = 16
NW = NC * NS
N_PER_W = B // NW
CHUNK = 4096
NCHUNK = N_PER_W // CHUNK
LANES = 16

MAGIC = 12582912.0


def _scale_round(x, scale, hi):
    t = x * jnp.float32(scale) + jnp.float32(scale + MAGIC)
    i = t.astype(jnp.int32) - jnp.int32(MAGIC)
    return jnp.minimum(jnp.maximum(i, 0), hi)


def _make_kernel():
    mesh = plsc.VectorSubcoreMesh(core_axis_name="c", subcore_axis_name="s")

    @functools.partial(
        pl.kernel,
        mesh=mesh,
        out_type=jax.ShapeDtypeStruct((B,), jnp.float32),
        scratch_types=[
            pltpu.VMEM((2 * N_PER_W,), jnp.float32),
            pltpu.VMEM((CHUNK,), jnp.int32),
            pltpu.VMEM((CHUNK,), jnp.int32),
            pltpu.VMEM((CHUNK,), jnp.int32),
            pltpu.VMEM((CHUNK,), jnp.int32),
            pltpu.VMEM((N_PER_W,), jnp.float32),
            pltpu.SemaphoreType.DMA,
            pltpu.SemaphoreType.DMA,
            pltpu.SemaphoreType.DMA,
            pltpu.SemaphoreType.DMA,
            pltpu.SemaphoreType.DMA,
            pltpu.SemaphoreType.DMA,
        ],
        compiler_params=pltpu.CompilerParams(
            needs_layout_passes=False, disable_bounds_checks=True,
            skip_device_barrier=True),
    )
    def night_light(x_hbm, f_hbm, y_hbm,
                    xv, ix0, ix1, ix2, ix3, yall,
                    sx, sg0, sg1, sg2, sg3, sw):
        wid = lax.axis_index("s") * NC + lax.axis_index("c")
        base0 = wid * N_PER_W
        ix = [ix0, ix1, ix2, ix3]
        sg = [sg0, sg1, sg2, sg3]

        hx = pltpu.async_copy(
            x_hbm.at[pl.ds(2 * base0, 2 * N_PER_W)], xv, sx)

        def compute(k):
            b = k % 4

            def pt_body(i, c2):
                j = k * CHUNK // LANES + i
                off = (j >> 3) * 256 + (j & 7) * LANES
                x0 = xv[pl.ds(off, LANES)]
                x1 = xv[pl.ds(off + 128, LANES)]
                col = _scale_round(x0, W // 2, W - 1)
                row = _scale_round(x1, H // 2, H - 1)
                p = ((row << 7) + (row >> 3) * 64512
                     + col + (col >> 7) * 896)
                ix[b][pl.ds(i * LANES, LANES)] = p
                return c2

            lax.fori_loop(0, CHUNK // LANES, pt_body, 0, unroll=4)

        def gather(k):
            return pltpu.async_copy(
                f_hbm.at[ix[k % 4]],
                yall.at[pl.ds(k * CHUNK, CHUNK)],
                sg[k % 4])

        hx.wait()
        hg = {}
        for k in range(NCHUNK):
            compute(k)
            if k >= 3:
                hg[k - 3].wait()
            hg[k] = gather(k)
        for k in range(NCHUNK - 3, NCHUNK):
            hg[k].wait()
        pltpu.async_copy(yall, y_hbm.at[pl.ds(base0, N_PER_W)], sw).wait()

    return night_light


_night_light = _make_kernel()


@jax.jit
def kernel(x, f):
    x_phys = x.reshape(B // 128, 128, 2).transpose(0, 2, 1).reshape(2 * B)
    f_phys = (
        f.reshape(H // 8, 8, W // 128, 128)
        .transpose(0, 2, 1, 3)
        .reshape(H * W)
    )
    return _night_light(x_phys, f_phys)


# speedup vs baseline: 1.0704x; 1.0204x over previous
import functools

import jax
import jax.numpy as jnp
from jax import lax
from jax.experimental import pallas as pl
from jax.experimental.pallas import tpu as pltpu
from jax.experimental.pallas import tpu_sc as plsc

H = 4096
W = 8192
B = 1048576
NC = 2
NS = 16
NW = NC * NS
N_PER_W = B // NW
CHUNK = 4096
NCHUNK = N_PER_W // CHUNK
LANES = 16
DEPTH = 3

MAGIC = 12582912.0


def _scale_round(x, scale, hi):
    t = x * jnp.float32(scale) + jnp.float32(scale + MAGIC)
    i = t.astype(jnp.int32) - jnp.int32(MAGIC)
    return jnp.minimum(jnp.maximum(i, 0), hi)


def _make_kernel():
    mesh = plsc.VectorSubcoreMesh(core_axis_name="c", subcore_axis_name="s")

    @functools.partial(
        pl.kernel,
        mesh=mesh,
        out_type=jax.ShapeDtypeStruct((B,), jnp.float32),
        scratch_types=[
            pltpu.VMEM((2 * CHUNK,), jnp.float32),
            pltpu.VMEM((2 * CHUNK,), jnp.float32),
            pltpu.VMEM((CHUNK,), jnp.int32),
            pltpu.VMEM((CHUNK,), jnp.int32),
            pltpu.VMEM((CHUNK,), jnp.int32),
            pltpu.VMEM((CHUNK,), jnp.int32),
            pltpu.VMEM((N_PER_W,), jnp.float32),
            pltpu.SemaphoreType.DMA,
            pltpu.SemaphoreType.DMA,
            pltpu.SemaphoreType.DMA,
            pltpu.SemaphoreType.DMA,
            pltpu.SemaphoreType.DMA,
            pltpu.SemaphoreType.DMA,
            pltpu.SemaphoreType.DMA,
        ],
        compiler_params=pltpu.CompilerParams(
            needs_layout_passes=False, disable_bounds_checks=True),
    )
    def night_light(x_hbm, f_hbm, y_hbm,
                    xv0, xv1, ix0, ix1, ix2, ix3, yall,
                    sx0, sx1, sg0, sg1, sg2, sg3, sw):
        wid = lax.axis_index("s") * NC + lax.axis_index("c")
        base0 = wid * N_PER_W
        xv, sx = [xv0, xv1], [sx0, sx1]
        ix = [ix0, ix1, ix2, ix3]
        sg = [sg0, sg1, sg2, sg3]

        def xcopy(k):
            return pltpu.async_copy(
                x_hbm.at[pl.ds(2 * (base0 + k * CHUNK), 2 * CHUNK)],
                xv[k % 2], sx[k % 2])

        def compute(k):
            b, xb = k % 4, k % 2

            def pt_body(i, c2):
                off = (i >> 3) * 256 + (i & 7) * LANES
                x0 = xv[xb][pl.ds(off, LANES)]
                x1 = xv[xb][pl.ds(off + 128, LANES)]
                col = _scale_round(x0, W // 2, W - 1)
                row = _scale_round(x1, H // 2, H - 1)
                p = ((row << 7) + (row >> 3) * 64512
                     + col + (col >> 7) * 896)
                ix[b][pl.ds(i * LANES, LANES)] = p
                return c2

            lax.fori_loop(0, CHUNK // LANES, pt_body, 0, unroll=4)

        def gather(k):
            return pltpu.async_copy(
                f_hbm.at[ix[k % 4]],
                yall.at[pl.ds(k * CHUNK, CHUNK)],
                sg[k % 4])

        def wb(k):
            return pltpu.async_copy(
                yall.at[pl.ds(k * CHUNK, CHUNK)],
                y_hbm.at[pl.ds(base0 + k * CHUNK, CHUNK)],
                sw)

        hx, hg, hw = {}, {}, {}
        hx[0] = xcopy(0)
        for k in range(NCHUNK):
            hx[k].wait()
            if k + 1 < NCHUNK:
                hx[k + 1] = xcopy(k + 1)
            compute(k)
            if k >= DEPTH:
                hg[k - DEPTH].wait()
                hw[k - DEPTH] = wb(k - DEPTH)
            hg[k] = gather(k)
        for k in range(NCHUNK - DEPTH, NCHUNK):
            hg[k].wait()
            hw[k] = wb(k)
        for k in range(NCHUNK):
            hw[k].wait()

    return night_light


_night_light = _make_kernel()


@jax.jit
def kernel(x, f):
    x_phys = x.reshape(B // 128, 128, 2).transpose(0, 2, 1).reshape(2 * B)
    f_phys = (
        f.reshape(H // 8, 8, W // 128, 128)
        .transpose(0, 2, 1, 3)
        .reshape(H * W)
    )
    return _night_light(x_phys, f_phys)
